# baseline (speedup 1.0000x reference)
import jax
import jax.numpy as jnp
from jax.experimental import pallas as pl

VOCAB = 6
HIDDEN = 768
BATCH = 4
SEQ = 8192
BLK_SEQ = 1024


def _pad_kernel(in_ref, out_ref):
    out_ref[...] = jnp.zeros_like(out_ref)
    out_ref[:, :, 0:VOCAB] = in_ref[...]


def kernel(input_probs):
    grid = (BATCH, SEQ // BLK_SEQ)
    return pl.pallas_call(
        _pad_kernel,
        grid=grid,
        in_specs=[pl.BlockSpec((1, BLK_SEQ, VOCAB), lambda i, j: (i, j, 0))],
        out_specs=pl.BlockSpec((1, BLK_SEQ, HIDDEN), lambda i, j: (i, j, 0)),
        out_shape=jax.ShapeDtypeStruct((BATCH, SEQ, HIDDEN), input_probs.dtype),
    )(input_probs)


# speedup vs baseline: 1.1412x; 1.1412x over previous
import jax
import jax.numpy as jnp
from jax.experimental import pallas as pl

VOCAB = 6
HIDDEN = 768
ROWS = 4 * 8192
BLK = 2048


def _pad_kernel(in_ref, out_ref):
    x = in_ref[...]
    first = jnp.concatenate(
        [x, jnp.zeros((BLK, 128 - VOCAB), x.dtype)], axis=-1)
    out_ref[:, 0:128] = first
    out_ref[:, 128:] = jnp.zeros((BLK, HIDDEN - 128), x.dtype)


def kernel(input_probs):
    flat = input_probs.reshape(ROWS, VOCAB)
    out = pl.pallas_call(
        _pad_kernel,
        grid=(ROWS // BLK,),
        in_specs=[pl.BlockSpec((BLK, VOCAB), lambda i: (i, 0))],
        out_specs=pl.BlockSpec((BLK, HIDDEN), lambda i: (i, 0)),
        out_shape=jax.ShapeDtypeStruct((ROWS, HIDDEN), input_probs.dtype),
    )(flat)
    return out.reshape(input_probs.shape[0], input_probs.shape[1], HIDDEN)


# speedup vs baseline: 1.1631x; 1.0192x over previous
import jax
import jax.numpy as jnp
from jax.experimental import pallas as pl

VOCAB = 6
HIDDEN = 768
ROWS = 4 * 8192
BLK = 4096


def _pad_kernel(in_ref, out_ref):
    x = in_ref[...]
    first = jnp.concatenate(
        [x, jnp.zeros((BLK, 128 - VOCAB), x.dtype)], axis=-1)
    out_ref[:, 0:128] = first
    out_ref[:, 128:] = jnp.zeros((BLK, HIDDEN - 128), x.dtype)


def kernel(input_probs):
    flat = input_probs.reshape(ROWS, VOCAB)
    out = pl.pallas_call(
        _pad_kernel,
        grid=(ROWS // BLK,),
        in_specs=[pl.BlockSpec((BLK, VOCAB), lambda i: (i, 0))],
        out_specs=pl.BlockSpec((BLK, HIDDEN), lambda i: (i, 0)),
        out_shape=jax.ShapeDtypeStruct((ROWS, HIDDEN), input_probs.dtype),
    )(flat)
    return out.reshape(input_probs.shape[0], input_probs.shape[1], HIDDEN)
